# baseline (speedup 1.0000x reference)
import functools

import jax
import jax.numpy as jnp
import numpy as np
from jax import lax
from jax.experimental import pallas as pl
from jax.experimental.pallas import tpu as pltpu
from jax.experimental.pallas import tpu_sc as plsc

N = 10000
E = 320000
HID = 128
TDIM = 32
NG = 16
NRES = 20

NWORK = 32
CHUNK = 128
EC = E // CHUNK
ROWS_PER_SUB = 1000
ZROWS = 125

_f32 = jnp.float32


def _silu(v):
    return v * jax.nn.sigmoid(v)


def _sc_mesh():
    return plsc.VectorSubcoreMesh(core_axis_name="c", subcore_axis_name="s")


NC0 = EC // NWORK
NEXTRA = EC - NC0 * NWORK


def _worker_id():
    c = lax.axis_index("c")
    s = lax.axis_index("s")
    return s * 2 + c


def _gather_pair(table_a, table_b, src, dst):
    def impl(a_hbm, b_hbm, s_hbm, d_hbm, out_hbm,
             sidx, didx, abuf, bbuf, obuf, gsem_a, gsem_b, wsem):
        w = _worker_id()

        def start_gather(i, b):
            off = (w + i * NWORK) * CHUNK
            pltpu.sync_copy(s_hbm.at[pl.ds(off, CHUNK)], sidx[b])
            pltpu.sync_copy(d_hbm.at[pl.ds(off, CHUNK)], didx[b])
            pltpu.make_async_copy(a_hbm.at[sidx[b]], abuf[b], gsem_a[b]).start()
            pltpu.make_async_copy(b_hbm.at[didx[b]], bbuf[b], gsem_b[b]).start()

        def finish_chunk(i, b):
            pltpu.make_async_copy(a_hbm.at[sidx[b]], abuf[b], gsem_a[b]).wait()
            pltpu.make_async_copy(b_hbm.at[didx[b]], bbuf[b], gsem_b[b]).wait()
            pltpu.make_async_copy(
                obuf[b], out_hbm.at[pl.ds(w * CHUNK, CHUNK), :], wsem[b]
            ).wait()

            def addrow(r, carry2):
                for j in range(HID // 16):
                    sl = pl.ds(j * 16, 16)
                    obuf[b][r, sl] = abuf[b][r, sl] + bbuf[b][r, sl]
                return carry2

            lax.fori_loop(0, CHUNK, addrow, 0)
            off = (w + i * NWORK) * CHUNK
            pltpu.make_async_copy(
                obuf[b], out_hbm.at[pl.ds(off, CHUNK), :], wsem[b]
            ).start()

        for b in range(2):
            start_gather(b, b)
            pltpu.make_async_copy(
                obuf[b], out_hbm.at[pl.ds(w * CHUNK, CHUNK), :], wsem[b]
            ).start()

        def body(i, carry):
            for b in range(2):
                @pl.when(i % 2 == b)
                def _():
                    finish_chunk(i, b)
                    start_gather(i + 2, b)
            return carry

        lax.fori_loop(0, NC0 - 2, body, 0)
        finish_chunk(NC0 - 2, (NC0 - 2) % 2)
        finish_chunk(NC0 - 1, (NC0 - 1) % 2)

        @pl.when(w < NEXTRA)
        def _extra():
            b = NC0 % 2
            start_gather(NC0, b)
            finish_chunk(NC0, b)

        for b in range(2):
            pltpu.make_async_copy(
                obuf[b], out_hbm.at[pl.ds(w * CHUNK, CHUNK), :], wsem[b]
            ).wait()

    k = functools.partial(
        pl.kernel,
        out_type=jax.ShapeDtypeStruct((E, HID), _f32),
        mesh=_sc_mesh(),
        scratch_types=[
            [pltpu.VMEM((CHUNK,), jnp.int32)] * 2,
            [pltpu.VMEM((CHUNK,), jnp.int32)] * 2,
            [pltpu.VMEM((CHUNK, HID), _f32)] * 2,
            [pltpu.VMEM((CHUNK, HID), _f32)] * 2,
            [pltpu.VMEM((CHUNK, HID), _f32)] * 2,
            [pltpu.SemaphoreType.DMA] * 2,
            [pltpu.SemaphoreType.DMA] * 2,
            [pltpu.SemaphoreType.DMA] * 2,
        ],
    )(impl)
    return k(table_a, table_b, src, dst)


def _rel_gather(nposp, posp, src, dst):
    def impl(a_hbm, b_hbm, s_hbm, d_hbm, rel_hbm,
             sidx, didx, abuf, bbuf, gsem_a, gsem_b):
        w = _worker_id()

        def chunk(i, carry):
            off = (w + i * NWORK) * CHUNK
            pltpu.sync_copy(s_hbm.at[pl.ds(off, CHUNK)], sidx)
            pltpu.sync_copy(d_hbm.at[pl.ds(off, CHUNK)], didx)
            cp1 = pltpu.make_async_copy(a_hbm.at[sidx], abuf, gsem_a)
            cp1.start()
            cp2 = pltpu.make_async_copy(b_hbm.at[didx], bbuf, gsem_b)
            cp2.start()
            cp1.wait()
            cp2.wait()

            def row(r, carry2):
                abuf[r, pl.ds(0, 16)] = abuf[r, pl.ds(0, 16)] + bbuf[r, pl.ds(0, 16)]
                return carry2

            lax.fori_loop(0, CHUNK, row, 0)
            pltpu.sync_copy(abuf, rel_hbm.at[pl.ds(off, CHUNK), :])
            return carry

        lax.fori_loop(0, NC0, chunk, 0)

        @pl.when(w < NEXTRA)
        def _extra():
            chunk(NC0, 0)

    k = functools.partial(
        pl.kernel,
        out_type=jax.ShapeDtypeStruct((E, 16), _f32),
        mesh=_sc_mesh(),
        compiler_params=pltpu.CompilerParams(use_tc_tiling_on_sc=False),
        scratch_types=[
            pltpu.VMEM((CHUNK,), jnp.int32),
            pltpu.VMEM((CHUNK,), jnp.int32),
            pltpu.VMEM((CHUNK, 16), _f32),
            pltpu.VMEM((CHUNK, 16), _f32),
            pltpu.SemaphoreType.DMA,
            pltpu.SemaphoreType.DMA,
        ],
    )(impl)
    return k(nposp, posp, src, dst)


def _segment_scatter(rows, dst, width):
    def impl(m_hbm, d_hbm, out_hbm, didx, mbuf, agg_sh, lsem, ssem):
        c = lax.axis_index("c")
        s = lax.axis_index("s")
        w = _worker_id()

        def zrow(r, carry):
            for j in range(width // 16):
                mbuf[0][r, pl.ds(j * 16, 16)] = jnp.zeros((16,), _f32)
            return carry

        lax.fori_loop(0, ZROWS, zrow, 0)

        @pl.when(s < 10)
        def _zero():
            for kk in range(ROWS_PER_SUB // ZROWS):
                pltpu.sync_copy(
                    mbuf[0].at[pl.ds(0, ZROWS), :],
                    agg_sh.at[pl.ds(s * ROWS_PER_SUB + kk * ZROWS, ZROWS), :],
                )

        plsc.subcore_barrier()

        def start_load(i, b):
            off = (w + i * NWORK) * CHUNK
            pltpu.sync_copy(d_hbm.at[pl.ds(off, CHUNK)], didx[b])
            pltpu.make_async_copy(
                m_hbm.at[pl.ds(off, CHUNK), :], mbuf[b], lsem[b]
            ).start()

        def wait_load(b):
            pltpu.make_async_copy(
                m_hbm.at[pl.ds(w * CHUNK, CHUNK), :], mbuf[b], lsem[b]
            ).wait()

        def start_scatter(b):
            pltpu.make_async_copy(mbuf[b], agg_sh.at[didx[b]], ssem[b]).start(add=True)

        def wait_scatter(b):
            pltpu.make_async_copy(mbuf[b], agg_sh.at[didx[b]], ssem[b]).wait()

        def proc(i, b, first):
            if not first:
                wait_scatter(b)
            start_load(i, b)
            wait_load(b)
            start_scatter(b)

        def body(i, carry):
            for b in range(2):
                @pl.when(i % 2 == b)
                def _():
                    @pl.when(i >= 2)
                    def _w():
                        wait_scatter(b)

                    start_load(i, b)
                    wait_load(b)
                    start_scatter(b)
            return carry

        lax.fori_loop(0, NC0, body, 0)

        @pl.when(w < NEXTRA)
        def _extra():
            proc(NC0, NC0 % 2, False)

        for b in range(2):
            wait_scatter(b)

        plsc.subcore_barrier()

        @pl.when(s < 10)
        def _copy_out():
            r0 = s * ROWS_PER_SUB
            pltpu.sync_copy(
                agg_sh.at[pl.ds(r0, ROWS_PER_SUB), :],
                out_hbm.at[c, pl.ds(r0, ROWS_PER_SUB), :],
            )

    k = functools.partial(
        pl.kernel,
        out_type=jax.ShapeDtypeStruct((2, N, width), _f32),
        mesh=_sc_mesh(),
        compiler_params=pltpu.CompilerParams(use_tc_tiling_on_sc=(width == HID)),
        scratch_types=[
            [pltpu.VMEM((CHUNK,), jnp.int32)] * 2,
            [pltpu.VMEM((CHUNK, width), _f32)] * 2,
            pltpu.VMEM_SHARED((N, width), _f32),
            [pltpu.SemaphoreType.DMA] * 2,
            [pltpu.SemaphoreType.DMA] * 2,
        ],
    )(impl)
    return k(rows, dst)


_NB = 1000
_EB = 512


def _w_spec(shape):
    return pl.BlockSpec(shape, lambda i: (0,) * len(shape))


def _init_nodes(x2, b2, emb, te, iW1h, iW1t, ib1, iW2, ib2):
    def body(x_ref, b_ref, emb_ref, te_ref, w1h_ref, w1t_ref, b1_ref, w2_ref, b2_ref, o_ref):
        xv = x_ref[...]
        bv = b_ref[...]
        h0 = jnp.zeros((_NB, HID), _f32)
        for r in range(NRES):
            h0 = jnp.where(xv == r, emb_ref[pl.ds(r, 1), :], h0)
        ht = jnp.zeros((_NB, HID), _f32)
        for r in range(NG):
            ht = jnp.where(bv == r, te_ref[pl.ds(r, 1), :], ht)
        p = (
            jnp.dot(h0, w1h_ref[...], preferred_element_type=_f32)
            + jnp.dot(ht, w1t_ref[...], preferred_element_type=_f32)
            + b1_ref[...]
        )
        o_ref[...] = jnp.dot(_silu(p), w2_ref[...], preferred_element_type=_f32) + b2_ref[...]

    return pl.pallas_call(
        body,
        grid=(N // _NB,),
        in_specs=[
            pl.BlockSpec((_NB, 1), lambda i: (i, 0)),
            pl.BlockSpec((_NB, 1), lambda i: (i, 0)),
            _w_spec((NRES, HID)),
            _w_spec((NG, HID)),
            _w_spec((HID, HID)),
            _w_spec((HID, HID)),
            _w_spec((1, HID)),
            _w_spec((HID, HID)),
            _w_spec((1, HID)),
        ],
        out_specs=pl.BlockSpec((_NB, HID), lambda i: (i, 0)),
        out_shape=jax.ShapeDtypeStruct((N, HID), _f32),
    )(x2, b2, emb, te, iW1h, iW1t, ib1, iW2, ib2)


def _ab_tables(h, Ws, Wd, eb1):
    def body(h_ref, ws_ref, wd_ref, b1_ref, a_ref, b_ref):
        hv = h_ref[...]
        a_ref[...] = jnp.dot(hv, ws_ref[...], preferred_element_type=_f32) + b1_ref[...]
        b_ref[...] = jnp.dot(hv, wd_ref[...], preferred_element_type=_f32)

    return pl.pallas_call(
        body,
        grid=(N // _NB,),
        in_specs=[
            pl.BlockSpec((_NB, HID), lambda i: (i, 0)),
            _w_spec((HID, HID)),
            _w_spec((HID, HID)),
            _w_spec((1, HID)),
        ],
        out_specs=[
            pl.BlockSpec((_NB, HID), lambda i: (i, 0)),
            pl.BlockSpec((_NB, HID), lambda i: (i, 0)),
        ],
        out_shape=[
            jax.ShapeDtypeStruct((N, HID), _f32),
            jax.ShapeDtypeStruct((N, HID), _f32),
        ],
    )(h, Ws, Wd, eb1)


def _prep_d2(rel):
    def body(rel_ref, o_ref):
        i = pl.program_id(0)
        relb = rel_ref[...]
        d2 = jnp.sum(relb * relb, axis=1, keepdims=True)
        di = lax.bitcast_convert_type(d2, jnp.int32)
        lsb = lax.shift_right_logical(di, 16) & 1
        di = (di + 32767 + lsb) & jnp.int32(-65536)
        d2b = lax.bitcast_convert_type(di, _f32)
        o_ref[pl.ds(i * (_EB // CHUNK), _EB // CHUNK), :] = d2b.reshape(
            _EB // CHUNK, CHUNK
        )

    return pl.pallas_call(
        body,
        grid=(E // _EB,),
        in_specs=[pl.BlockSpec((_EB, 16), lambda i: (i, 0))],
        out_specs=_w_spec((EC, CHUNK)),
        out_shape=jax.ShapeDtypeStruct((EC, CHUNK), _f32),
    )(rel)


def _edge_mlp(pre, d2pk, wdb, spk_in, rel, eW2, eb2, cWT, cb, last):
    bs = _EB // CHUNK

    def body(pre_ref, d2_ref, wd_ref, ss_ref, *rest):
        if last:
            rel_ref, w2_ref, b2_ref, cwt_ref, cb_ref, out0 = rest
        else:
            w2_ref, b2_ref, cwt_ref, cb_ref, out0, out1 = rest
        i = pl.program_id(0)
        rows = pl.ds(i * bs, bs)
        d2t = jnp.transpose(d2_ref[rows, :])
        wd = wd_ref[...].astype(_f32)
        p = jnp.concatenate(
            [
                pre_ref[pl.ds(a * CHUNK, CHUNK), :] + d2t[:, a : a + 1] * wd
                for a in range(bs)
            ],
            axis=0,
        )
        m1 = _silu(p)
        q = jnp.dot(m1, w2_ref[...], preferred_element_type=_f32) + b2_ref[...]
        m = _silu(q)
        s = jnp.sum(m * cwt_ref[...], axis=1, keepdims=True) + cb_ref[...]
        spk_new = ss_ref[rows, :] + s.reshape(bs, CHUNK)
        if last:
            st = jnp.transpose(spk_new)
            for a in range(bs):
                out0[pl.ds(a * CHUNK, CHUNK), :] = (
                    rel_ref[pl.ds(a * CHUNK, CHUNK), :] * st[:, a : a + 1]
                )
        else:
            out0[...] = m
            out1[rows, :] = spk_new

    spk_spec = _w_spec((EC, CHUNK))
    in_specs = [
        pl.BlockSpec((_EB, HID), lambda i: (i, 0)),
        spk_spec,
        _w_spec((1, HID)),
        spk_spec,
    ]
    if last:
        in_specs.append(pl.BlockSpec((_EB, 16), lambda i: (i, 0)))
        out_specs = pl.BlockSpec((_EB, 16), lambda i: (i, 0))
        out_shape = jax.ShapeDtypeStruct((E, 16), _f32)
    else:
        out_specs = [pl.BlockSpec((_EB, HID), lambda i: (i, 0)), spk_spec]
        out_shape = [
            jax.ShapeDtypeStruct((E, HID), _f32),
            jax.ShapeDtypeStruct((EC, CHUNK), _f32),
        ]
    in_specs += [
        _w_spec((HID, HID)),
        _w_spec((1, HID)),
        _w_spec((1, HID)),
        _w_spec((1, 1)),
    ]
    args = (pre, d2pk, wdb, spk_in) + ((rel,) if last else ()) + (eW2, eb2, cWT, cb)
    return pl.pallas_call(
        body,
        grid=(E // _EB,),
        in_specs=in_specs,
        out_specs=out_specs,
        out_shape=out_shape,
    )(*args)


def _node_update(h, agg2, nW1h, nW1a, nb1, nW2, nb2):
    def body(h_ref, agg_ref, w1h_ref, w1a_ref, b1_ref, w2_ref, b2_ref, o_ref):
        agg = agg_ref[0] + agg_ref[1]
        hv = h_ref[...]
        p = (
            jnp.dot(hv, w1h_ref[...], preferred_element_type=_f32)
            + jnp.dot(agg, w1a_ref[...], preferred_element_type=_f32)
            + b1_ref[...]
        )
        o_ref[...] = hv + jnp.dot(_silu(p), w2_ref[...], preferred_element_type=_f32) + b2_ref[...]

    return pl.pallas_call(
        body,
        grid=(N // _NB,),
        in_specs=[
            pl.BlockSpec((_NB, HID), lambda i: (i, 0)),
            pl.BlockSpec((2, _NB, HID), lambda i: (0, i, 0)),
            _w_spec((HID, HID)),
            _w_spec((HID, HID)),
            _w_spec((1, HID)),
            _w_spec((HID, HID)),
            _w_spec((1, HID)),
        ],
        out_specs=pl.BlockSpec((_NB, HID), lambda i: (i, 0)),
        out_shape=jax.ShapeDtypeStruct((N, HID), _f32),
    )(h, agg2, nW1h, nW1a, nb1, nW2, nb2)


def _final_ln(tp2, ln_w, ln_b):
    def body(tp_ref, w_ref, b_ref, o_ref):
        t3 = (tp_ref[0] + tp_ref[1])[:, :3]
        mu = jnp.mean(t3, axis=1, keepdims=True)
        var = jnp.mean((t3 - mu) * (t3 - mu), axis=1, keepdims=True)
        o_ref[...] = (t3 - mu) * lax.rsqrt(var + 1e-5) * w_ref[...] + b_ref[...]

    return pl.pallas_call(
        body,
        grid=(N // _NB,),
        in_specs=[
            pl.BlockSpec((2, _NB, 16), lambda i: (0, i, 0)),
            _w_spec((1, 3)),
            _w_spec((1, 3)),
        ],
        out_specs=pl.BlockSpec((_NB, 3), lambda i: (i, 0)),
        out_shape=jax.ShapeDtypeStruct((N, 3), _f32),
    )(tp2, ln_w, ln_b)


def _time_embed_small(t, p):
    half = TDIM // 2
    freq = jnp.exp(jnp.arange(half, dtype=_f32) * (-np.log(10000.0) / (half - 1)))
    e = t[:, None] * freq[None, :]
    te = jnp.concatenate([jnp.sin(e), jnp.cos(e)], axis=-1)
    return _silu(te @ p["tW1"] + p["tb1"]) @ p["tW2"] + p["tb2"]


def _bf16_round_via_ints(a):
    ai = lax.bitcast_convert_type(a, jnp.int32)
    lsb = lax.shift_right_logical(ai, 16) & 1
    ai = (ai + 32767 + lsb) & jnp.int32(-65536)
    return lax.bitcast_convert_type(ai, _f32)


def kernel(x, pos, batch, t, edge_index, params):
    p = params
    src = edge_index[0].astype(jnp.int32)
    dst = edge_index[1].astype(jnp.int32)
    x2 = x.astype(jnp.int32).reshape(N, 1)
    b2 = batch.astype(jnp.int32).reshape(N, 1)

    te = _time_embed_small(t, p)

    h = _init_nodes(
        x2, b2, p["emb"], te,
        p["iW1"][:HID], p["iW1"][HID:], p["ib1"].reshape(1, HID),
        p["iW2"], p["ib2"].reshape(1, HID),
    )

    posp = jnp.pad(pos, ((0, 0), (0, 13)))
    rel = _rel_gather(-posp, posp, src, dst)
    d2pk = _prep_d2(rel)

    spk = jnp.zeros((EC, CHUNK), _f32)
    rsp = None
    for li, lp in enumerate(p["layers"]):
        last = li == len(p["layers"]) - 1
        A, B = _ab_tables(h, lp["eW1"][:HID], lp["eW1"][HID : 2 * HID], lp["eb1"].reshape(1, HID))
        wdb = lp["eW1"][2 * HID].reshape(1, HID).astype(jnp.bfloat16)
        pre = _gather_pair(A, B, src, dst)
        ew = (
            lp["eW2"],
            lp["eb2"].reshape(1, HID),
            lp["cW"].reshape(1, HID),
            lp["cb"].reshape(1, 1),
        )
        if last:
            rsp = _edge_mlp(pre, d2pk, wdb, spk, rel, *ew, True)
        else:
            m, spk = _edge_mlp(pre, d2pk, wdb, spk, None, *ew, False)
            agg2 = _segment_scatter(m, dst, HID)
            h = _node_update(
                h, agg2,
                lp["nW1"][:HID], lp["nW1"][HID:], lp["nb1"].reshape(1, HID),
                lp["nW2"], lp["nb2"].reshape(1, HID),
            )

    tp2 = _segment_scatter(rsp, dst, 16)
    return _final_ln(tp2, p["ln_w"].reshape(1, 3), p["ln_b"].reshape(1, 3))
